# baseline (speedup 1.0000x reference)
import functools

import jax
import jax.numpy as jnp
from jax import lax
from jax.experimental import pallas as pl
from jax.experimental.pallas import tpu as pltpu
from jax.experimental.pallas import tpu_sc as plsc

VOCAB = 1000000
DIM = 64
B = 16384
K = 64

_NC, _NS = 2, 16
_NW = _NC * _NS
_BPW = B // _NW
_CHUNK = 128
_NCH = _BPW // _CHUNK


def _make_gather():
  mesh = plsc.VectorSubcoreMesh(core_axis_name="c", subcore_axis_name="s",
                                num_cores=_NC)

  @functools.partial(
      pl.kernel,
      mesh=mesh,
      compiler_params=pltpu.CompilerParams(use_tc_tiling_on_sc=False),
      out_type=[
          jax.ShapeDtypeStruct((B, DIM), jnp.float32),
          jax.ShapeDtypeStruct((B, DIM), jnp.float32),
          jax.ShapeDtypeStruct((K, DIM), jnp.float32),
      ],
      scratch_types=[
          pltpu.VMEM((_NCH, _CHUNK), jnp.int32),
          pltpu.VMEM((_NCH, _CHUNK), jnp.int32),
          pltpu.VMEM((_BPW, DIM), jnp.float32),
          pltpu.VMEM((_BPW, DIM), jnp.float32),
          pltpu.VMEM((K,), jnp.int32),
          pltpu.VMEM((K, DIM), jnp.float32),
          pltpu.SemaphoreType.DMA,
          pltpu.SemaphoreType.DMA,
          pltpu.SemaphoreType.DMA,
      ],
  )
  def gather_k(center_hbm, context_hbm, neg_hbm, in_emb_hbm, out_emb_hbm,
               cvec_hbm, xvec_hbm, nvec_hbm,
               cidx_v, xidx_v, crows_v, xrows_v, nidx_v, nrows_v,
               sem_a, sem_b, sem_n):
    wid = lax.axis_index("s") * _NC + lax.axis_index("c")
    base = wid * _BPW

    pltpu.sync_copy(center_hbm.at[wid], cidx_v)
    pltpu.sync_copy(context_hbm.at[wid], xidx_v)

    cps_a = [
        pltpu.async_copy(in_emb_hbm.at[cidx_v.at[j]],
                         crows_v.at[pl.ds(j * _CHUNK, _CHUNK)], sem_a)
        for j in range(_NCH)
    ]
    cps_b = [
        pltpu.async_copy(out_emb_hbm.at[xidx_v.at[j]],
                         xrows_v.at[pl.ds(j * _CHUNK, _CHUNK)], sem_b)
        for j in range(_NCH)
    ]

    @pl.when(wid == 0)
    def _():
      pltpu.sync_copy(neg_hbm, nidx_v)
      pltpu.async_copy(out_emb_hbm.at[nidx_v], nrows_v, sem_n).wait()
      pltpu.sync_copy(nrows_v, nvec_hbm)

    for cp in cps_a:
      cp.wait()
    pltpu.sync_copy(crows_v, cvec_hbm.at[pl.ds(base, _BPW)])
    for cp in cps_b:
      cp.wait()
    pltpu.sync_copy(xrows_v, xvec_hbm.at[pl.ds(base, _BPW)])

  return gather_k


_gather_cache = []


def _gather_fn():
  if not _gather_cache:
    _gather_cache.append(_make_gather())
  return _gather_cache[0]

_NB = 16
_BLK = B // _NB


def _softplus(x):
  return jnp.maximum(x, 0.0) + jnp.log1p(jnp.exp(-jnp.abs(x)))


def _score_body(cv_ref, xv_ref, neg_ref, out_ref, rows_v, acc_s):
  p = pl.program_id(0)
  j = pl.program_id(1)

  @pl.when(p == 0)
  def _():
    @pl.when(j == 0)
    def _():
      acc_s[0] = 0.0

    cv = cv_ref[...]
    xv = xv_ref[...]
    neg = neg_ref[...]
    pos = jnp.sum(cv * xv, axis=1)
    ns = lax.dot_general(cv, neg, (((1,), (1,)), ((), ())),
                         preferred_element_type=jnp.float32)
    acc_s[0] += jnp.sum(_softplus(ns))
    rows_v[pl.ds(j * _BLK, _BLK)] = _softplus(-pos)

  @pl.when(p == 1)
  def _():
    out_ref[...] = rows_v[pl.ds(j * _BLK, _BLK)] + acc_s[0]


def _score(cvec, xvec, nvec):
  return pl.pallas_call(
      _score_body,
      grid=(2, _NB),
      in_specs=[
          pl.BlockSpec((_BLK, DIM), lambda p, j: (j * (1 - p), 0)),
          pl.BlockSpec((_BLK, DIM), lambda p, j: (j * (1 - p), 0)),
          pl.BlockSpec((K, DIM), lambda p, j: (0, 0)),
      ],
      out_specs=pl.BlockSpec((_BLK,), lambda p, j: (j,)),
      out_shape=jax.ShapeDtypeStruct((B,), jnp.float32),
      scratch_shapes=[
          pltpu.VMEM((B,), jnp.float32),
          pltpu.SMEM((1,), jnp.float32),
      ],
  )(cvec, xvec, nvec)


def kernel(center, context, negatives, input_emb, output_emb):
  center_r = center.reshape(_NW, _NCH, _CHUNK)
  context_r = context.reshape(_NW, _NCH, _CHUNK)
  cvec, xvec, nvec = _gather_fn()(center_r, context_r, negatives,
                                  input_emb, output_emb)
  return _score(cvec, xvec, nvec)


# speedup vs baseline: 1.0049x; 1.0049x over previous
import functools

import jax
import jax.numpy as jnp
from jax import lax
from jax.experimental import pallas as pl
from jax.experimental.pallas import tpu as pltpu
from jax.experimental.pallas import tpu_sc as plsc

VOCAB = 1000000
DIM = 64
B = 16384
K = 64

_NC, _NS = 2, 16
_NW = _NC * _NS
_BPW = B // _NW
_CHUNK = 128
_NCH = _BPW // _CHUNK


def _sc_mesh():
  return plsc.VectorSubcoreMesh(core_axis_name="c", subcore_axis_name="s",
                                num_cores=_NC)


def _make_gather_center():
  @functools.partial(
      pl.kernel,
      mesh=_sc_mesh(),
      compiler_params=pltpu.CompilerParams(use_tc_tiling_on_sc=False),
      out_type=jax.ShapeDtypeStruct((B, DIM), jnp.float32),
      scratch_types=[
          pltpu.VMEM((_NCH, _CHUNK), jnp.int32),
          pltpu.VMEM((_BPW, DIM), jnp.float32),
          pltpu.SemaphoreType.DMA,
      ],
  )
  def gather_c(center_hbm, in_emb_hbm, cvec_hbm, cidx_v, crows_v, sem_a):
    wid = lax.axis_index("s") * _NC + lax.axis_index("c")
    base = wid * _BPW
    pltpu.sync_copy(center_hbm.at[wid], cidx_v)
    cps = [
        pltpu.async_copy(in_emb_hbm.at[cidx_v.at[j]],
                         crows_v.at[pl.ds(j * _CHUNK, _CHUNK)], sem_a)
        for j in range(_NCH)
    ]
    for cp in cps:
      cp.wait()
    pltpu.sync_copy(crows_v, cvec_hbm.at[pl.ds(base, _BPW)])

  return gather_c


def _make_gather_ctx():
  @functools.partial(
      pl.kernel,
      mesh=_sc_mesh(),
      compiler_params=pltpu.CompilerParams(use_tc_tiling_on_sc=False),
      out_type=[
          jax.ShapeDtypeStruct((B, DIM), jnp.float32),
          jax.ShapeDtypeStruct((K, DIM), jnp.float32),
      ],
      scratch_types=[
          pltpu.VMEM((_NCH, _CHUNK), jnp.int32),
          pltpu.VMEM((_BPW, DIM), jnp.float32),
          pltpu.VMEM((K,), jnp.int32),
          pltpu.VMEM((K, DIM), jnp.float32),
          pltpu.SemaphoreType.DMA,
          pltpu.SemaphoreType.DMA,
      ],
  )
  def gather_x(context_hbm, neg_hbm, out_emb_hbm, xvec_hbm, nvec_hbm,
               xidx_v, xrows_v, nidx_v, nrows_v, sem_b, sem_n):
    wid = lax.axis_index("s") * _NC + lax.axis_index("c")
    base = wid * _BPW
    pltpu.sync_copy(context_hbm.at[wid], xidx_v)
    cps = [
        pltpu.async_copy(out_emb_hbm.at[xidx_v.at[j]],
                         xrows_v.at[pl.ds(j * _CHUNK, _CHUNK)], sem_b)
        for j in range(_NCH)
    ]

    @pl.when(wid == 0)
    def _():
      pltpu.sync_copy(neg_hbm, nidx_v)
      pltpu.async_copy(out_emb_hbm.at[nidx_v], nrows_v, sem_n).wait()
      pltpu.sync_copy(nrows_v, nvec_hbm)

    for cp in cps:
      cp.wait()
    pltpu.sync_copy(xrows_v, xvec_hbm.at[pl.ds(base, _BPW)])

  return gather_x


_gather_cache = []


def _gather_fns():
  if not _gather_cache:
    _gather_cache.append((_make_gather_center(), _make_gather_ctx()))
  return _gather_cache[0]

_NB = 16
_BLK = B // _NB


def _softplus(x):
  return jnp.maximum(x, 0.0) + jnp.log1p(jnp.exp(-jnp.abs(x)))


def _score_body(cv_ref, xv_ref, neg_ref, out_ref, rows_v, acc_s):
  p = pl.program_id(0)
  j = pl.program_id(1)

  @pl.when(p == 0)
  def _():
    @pl.when(j == 0)
    def _():
      acc_s[0] = 0.0

    cv = cv_ref[...]
    xv = xv_ref[...]
    neg = neg_ref[...]
    pos = jnp.sum(cv * xv, axis=1)
    ns = lax.dot_general(cv, neg, (((1,), (1,)), ((), ())),
                         preferred_element_type=jnp.float32)
    acc_s[0] += jnp.sum(_softplus(ns))
    rows_v[pl.ds(j * _BLK, _BLK)] = _softplus(-pos)

  @pl.when(p == 1)
  def _():
    out_ref[...] = rows_v[pl.ds(j * _BLK, _BLK)] + acc_s[0]


def _score(cvec, xvec, nvec):
  return pl.pallas_call(
      _score_body,
      grid=(2, _NB),
      in_specs=[
          pl.BlockSpec((_BLK, DIM), lambda p, j: (j * (1 - p), 0)),
          pl.BlockSpec((_BLK, DIM), lambda p, j: (j * (1 - p), 0)),
          pl.BlockSpec((K, DIM), lambda p, j: (0, 0)),
      ],
      out_specs=pl.BlockSpec((_BLK,), lambda p, j: (j,)),
      out_shape=jax.ShapeDtypeStruct((B,), jnp.float32),
      scratch_shapes=[
          pltpu.VMEM((B,), jnp.float32),
          pltpu.SMEM((1,), jnp.float32),
      ],
  )(cvec, xvec, nvec)


def kernel(center, context, negatives, input_emb, output_emb):
  center_r = center.reshape(_NW, _NCH, _CHUNK)
  context_r = context.reshape(_NW, _NCH, _CHUNK)
  gather_c, gather_x = _gather_fns()
  cvec = gather_c(center_r, input_emb)
  xvec, nvec = gather_x(context_r, negatives, output_emb)
  return _score(cvec, xvec, nvec)
